# baseline (speedup 1.0000x reference)
import jax
import jax.numpy as jnp
from jax import lax
from jax.experimental import pallas as pl
from jax.experimental.pallas import tpu as pltpu

_TAU = 0.07
_Q, _K, _D = 2048, 16384, 256
_BQ = 128
_NQ = _Q // _BQ


def _bias_body(x_ref, b_ref):
    xx = x_ref[...]
    p = xx * xx
    p_hi = p.astype(jnp.bfloat16)
    p_lo = (p - p_hi.astype(jnp.float32)).astype(jnp.bfloat16)
    ones = jnp.ones((1, _D), jnp.bfloat16)
    dn = (((1,), (1,)), ((), ()))
    sqx = (
        lax.dot_general(ones, p_hi, dn, preferred_element_type=jnp.float32)
        + lax.dot_general(ones, p_lo, dn, preferred_element_type=jnp.float32)
    )
    b_ref[...] = sqx * (-1.0 / _TAU)


def _fused_body(y_ref, x_ref, b_ref, o_ref):
    dot = lax.dot_general(
        y_ref[...], x_ref[...], (((1,), (1,)), ((), ())),
        preferred_element_type=jnp.float32,
    )
    logits = dot * (2.0 / _TAU) + b_ref[...]

    m = jnp.max(logits, axis=1, keepdims=True)
    e = jnp.exp(logits - m)
    s = jnp.sum(e, axis=1, keepdims=True)
    o_ref[...] = e
    o_ref[...] = o_ref[...] * (1.0 / s)


def kernel(X, Y):
    bias = pl.pallas_call(
        _bias_body,
        grid=(1,),
        in_specs=[pl.BlockSpec((_K, _D), lambda i: (0, 0))],
        out_specs=pl.BlockSpec((1, _K), lambda i: (0, 0)),
        out_shape=jax.ShapeDtypeStruct((1, _K), jnp.float32),
    )(X)
    Xb = X.astype(jnp.bfloat16)
    Yb = Y.astype(jnp.bfloat16)
    return pl.pallas_call(
        _fused_body,
        grid=(_NQ,),
        in_specs=[
            pl.BlockSpec((_BQ, _D), lambda q: (q, 0)),
            pl.BlockSpec((_K, _D), lambda q: (0, 0)),
            pl.BlockSpec((1, _K), lambda q: (0, 0)),
        ],
        out_specs=pl.BlockSpec((_BQ, _K), lambda q: (q, 0)),
        out_shape=jax.ShapeDtypeStruct((_Q, _K), jnp.float32),
        compiler_params=pltpu.CompilerParams(
            dimension_semantics=("parallel",),
        ),
    )(Yb, Xb, bias)


# speedup vs baseline: 1.4128x; 1.4128x over previous
import jax
import jax.numpy as jnp
from jax import lax
from jax.experimental import pallas as pl
from jax.experimental.pallas import tpu as pltpu

_TAU = 0.07
_Q, _K, _D = 2048, 16384, 256
_BQ = 256
_NQ = _Q // _BQ


def _bias_body(x_ref, b_ref, xb_ref):
    xx = x_ref[...]
    xb_ref[...] = xx.astype(jnp.bfloat16)
    p = xx * xx
    p_hi = p.astype(jnp.bfloat16)
    p_lo = (p - p_hi.astype(jnp.float32)).astype(jnp.bfloat16)
    ones = jnp.ones((1, _D), jnp.bfloat16)
    dn = (((1,), (1,)), ((), ()))
    sqx = (
        lax.dot_general(ones, p_hi, dn, preferred_element_type=jnp.float32)
        + lax.dot_general(ones, p_lo, dn, preferred_element_type=jnp.float32)
    )
    b_ref[...] = sqx * (-1.0 / _TAU)


def _fused_body(y_ref, x_ref, b_ref, o_ref):
    dot = lax.dot_general(
        y_ref[...], x_ref[...], (((1,), (1,)), ((), ())),
        preferred_element_type=jnp.float32,
    )
    logits = dot * (2.0 / _TAU) + b_ref[...]

    m = jnp.max(logits, axis=1, keepdims=True)
    e = jnp.exp(logits - m)
    s = jnp.sum(e, axis=1, keepdims=True)
    o_ref[...] = e
    o_ref[...] = o_ref[...] * (1.0 / s)


def kernel(X, Y):
    bias, Xb = pl.pallas_call(
        _bias_body,
        grid=(1,),
        in_specs=[pl.BlockSpec((_K, _D), lambda i: (0, 0))],
        out_specs=[
            pl.BlockSpec((1, _K), lambda i: (0, 0)),
            pl.BlockSpec((_K, _D), lambda i: (0, 0)),
        ],
        out_shape=[
            jax.ShapeDtypeStruct((1, _K), jnp.float32),
            jax.ShapeDtypeStruct((_K, _D), jnp.bfloat16),
        ],
    )(X)
    Yb = Y.astype(jnp.bfloat16)
    return pl.pallas_call(
        _fused_body,
        grid=(_NQ,),
        in_specs=[
            pl.BlockSpec((_BQ, _D), lambda q: (q, 0)),
            pl.BlockSpec((_K, _D), lambda q: (0, 0)),
            pl.BlockSpec((1, _K), lambda q: (0, 0)),
        ],
        out_specs=pl.BlockSpec((_BQ, _K), lambda q: (q, 0)),
        out_shape=jax.ShapeDtypeStruct((_Q, _K), jnp.float32),
        compiler_params=pltpu.CompilerParams(
            dimension_semantics=("parallel",),
        ),
    )(Yb, Xb, bias)


# speedup vs baseline: 1.4495x; 1.0260x over previous
import jax
import jax.numpy as jnp
from jax import lax
from jax.experimental import pallas as pl
from jax.experimental.pallas import tpu as pltpu

_TAU = 0.07
_Q, _K, _D = 2048, 16384, 256
_BQ = 256
_NQ = _Q // _BQ


def _bias_body(x_ref, y_ref, b_ref, xb_ref, yb_ref):
    xx = x_ref[...]
    xb_ref[...] = xx.astype(jnp.bfloat16)
    yb_ref[...] = y_ref[...].astype(jnp.bfloat16)
    p = xx * xx
    p_hi = p.astype(jnp.bfloat16)
    p_lo = (p - p_hi.astype(jnp.float32)).astype(jnp.bfloat16)
    ones = jnp.ones((1, _D), jnp.bfloat16)
    dn = (((1,), (1,)), ((), ()))
    sqx = (
        lax.dot_general(ones, p_hi, dn, preferred_element_type=jnp.float32)
        + lax.dot_general(ones, p_lo, dn, preferred_element_type=jnp.float32)
    )
    b_ref[...] = sqx * (-1.0 / _TAU)


def _fused_body(y_ref, x_ref, b_ref, o_ref):
    dot = lax.dot_general(
        y_ref[...], x_ref[...], (((1,), (1,)), ((), ())),
        preferred_element_type=jnp.float32,
    )
    logits = dot * (2.0 / _TAU) + b_ref[...]

    m = jnp.max(logits, axis=1, keepdims=True)
    e = jnp.exp(logits - m)
    s = jnp.sum(e, axis=1, keepdims=True)
    o_ref[...] = e
    o_ref[...] = o_ref[...] * (1.0 / s)


def kernel(X, Y):
    bias, Xb, Yb = pl.pallas_call(
        _bias_body,
        grid=(1,),
        in_specs=[
            pl.BlockSpec((_K, _D), lambda i: (0, 0)),
            pl.BlockSpec((_Q, _D), lambda i: (0, 0)),
        ],
        out_specs=[
            pl.BlockSpec((1, _K), lambda i: (0, 0)),
            pl.BlockSpec((_K, _D), lambda i: (0, 0)),
            pl.BlockSpec((_Q, _D), lambda i: (0, 0)),
        ],
        out_shape=[
            jax.ShapeDtypeStruct((1, _K), jnp.float32),
            jax.ShapeDtypeStruct((_K, _D), jnp.bfloat16),
            jax.ShapeDtypeStruct((_Q, _D), jnp.bfloat16),
        ],
    )(X, Y)
    return pl.pallas_call(
        _fused_body,
        grid=(_NQ,),
        in_specs=[
            pl.BlockSpec((_BQ, _D), lambda q: (q, 0)),
            pl.BlockSpec((_K, _D), lambda q: (0, 0)),
            pl.BlockSpec((1, _K), lambda q: (0, 0)),
        ],
        out_specs=pl.BlockSpec((_BQ, _K), lambda q: (q, 0)),
        out_shape=jax.ShapeDtypeStruct((_Q, _K), jnp.float32),
        compiler_params=pltpu.CompilerParams(
            dimension_semantics=("parallel",),
        ),
    )(Yb, Xb, bias)
